# baseline (speedup 1.0000x reference)
import functools

import jax
import jax.numpy as jnp
from jax import lax
from jax.experimental import pallas as pl
from jax.experimental.pallas import tpu as pltpu
from jax.experimental.pallas import tpu_sc as plsc

D = 768
B = 4
T = 2048
N = B * T
EPS = 1e-6
NC, NS, L = 2, 16, 16
NW = NC * NS
TW = T // NW
C = 32
NCHUNK = (B * TW) // C
NJ = D // L
UNR = 8


def _rsqrt_scalar(a):
    i = lax.bitcast_convert_type(a, jnp.int32)
    i = jnp.int32(0x5F3759DF) - (i >> 1)
    y = lax.bitcast_convert_type(i, jnp.float32)
    half_a = 0.5 * a
    for _ in range(2):
        y = y * (1.5 - half_a * y * y)
    return y


_mesh = plsc.VectorSubcoreMesh(core_axis_name="c", subcore_axis_name="s")


@functools.partial(
    pl.kernel,
    mesh=_mesh,
    out_type=jax.ShapeDtypeStruct((N, D), jnp.float32),
    scratch_types=[
        pltpu.VMEM((3, C), jnp.int32),
        pltpu.VMEM((C, D), jnp.float32),
        pltpu.VMEM((C, D), jnp.float32),
        pltpu.VMEM((C, D), jnp.float32),
        pltpu.VMEM((C, D), jnp.float32),
        pltpu.VMEM((D,), jnp.float32),
        pltpu.SemaphoreType.DMA,
        pltpu.SemaphoreType.DMA,
        pltpu.SemaphoreType.DMA,
        pltpu.SemaphoreType.DMA,
        pltpu.SemaphoreType.DMA,
        pltpu.SemaphoreType.DMA,
    ],
)
def _emb_kernel(idx_hbm, ww_hbm, wp_hbm, sc_hbm, out_hbm,
                idx_v, tok0_v, tok1_v, tok2_v, pos_v, scale_v,
                g0_sem, g1_sem, g2_sem, o0_sem, o1_sem, o2_sem):
    wid = lax.axis_index("s") * NC + lax.axis_index("c")
    t0 = wid * TW
    toks = (tok0_v, tok1_v, tok2_v)
    gsems = (g0_sem, g1_sem, g2_sem)
    osems = (o0_sem, o1_sem, o2_sem)

    pltpu.sync_copy(sc_hbm, scale_v)
    pltpu.sync_copy(wp_hbm.at[pl.ds(t0, C)], pos_v)

    def flat0(c):
        return (c & (B - 1)) * T + t0 + (c >> 2) * C

    def start_gather(c):
        pltpu.sync_copy(idx_hbm.at[pl.ds(flat0(c), C)], idx_v.at[c % 3])
        return pltpu.async_copy(ww_hbm.at[idx_v.at[c % 3]], toks[c % 3],
                                gsems[c % 3])

    def compute(c):
        tok_v = toks[c % 3]

        @plsc.parallel_loop(0, C, 1)
        def row_body(r):
            zero = jnp.zeros((L,), jnp.float32)

            def p1(j, accs):
                accs = list(accs)
                for u in range(UNR):
                    sl = pl.ds(j * (UNR * L) + u * L, L)
                    x = tok_v[r, sl] + pos_v[r, sl]
                    tok_v[r, sl] = x
                    accs[u % 4] = accs[u % 4] + x * x
                return tuple(accs)

            a = lax.fori_loop(0, NJ // UNR, p1, (zero,) * 4)
            acc = (a[0] + a[1]) + (a[2] + a[3])
            lanes = [acc[l] for l in range(L)]
            while len(lanes) > 1:
                lanes = [lanes[i] + lanes[i + 1]
                         for i in range(0, len(lanes), 2)]
            ms = lanes[0] * (1.0 / D) + EPS
            rstd = jnp.full((L,), _rsqrt_scalar(ms), jnp.float32)

            def p2(j, cc):
                for u in range(UNR):
                    sl = pl.ds(j * (UNR * L) + u * L, L)
                    tok_v[r, sl] = tok_v[r, sl] * rstd * scale_v[sl]
                return cc

            lax.fori_loop(0, NJ // UNR, p2, 0)

    gathers = {0: start_gather(0)}
    outs = {}
    for c in range(NCHUNK):
        if c + 1 < NCHUNK:
            if c - 2 in outs:
                outs[c - 2].wait()
            gathers[c + 1] = start_gather(c + 1)
        gathers[c].wait()
        if c == B:
            pltpu.sync_copy(wp_hbm.at[pl.ds(t0 + C, C)], pos_v)
        compute(c)
        outs[c] = pltpu.async_copy(toks[c % 3],
                                   out_hbm.at[pl.ds(flat0(c), C)],
                                   osems[c % 3])
    for c in range(max(0, NCHUNK - 3), NCHUNK):
        outs[c].wait()


def kernel(idx, W_word, W_pos, rms_scale):
    out = _emb_kernel(idx.reshape(N), W_word, W_pos, rms_scale)
    return out.reshape(B, T, D)


# speedup vs baseline: 1.4532x; 1.4532x over previous
import functools

import jax
import jax.numpy as jnp
from jax import lax
from jax.experimental import pallas as pl
from jax.experimental.pallas import tpu as pltpu
from jax.experimental.pallas import tpu_sc as plsc

D = 768
B = 4
T = 2048
N = B * T
EPS = 1e-6
NC, NS, L = 2, 16, 16
NW = NC * NS
TW = T // NW
C = 32
NCHUNK = (B * TW) // C
NJ = D // L


def _rsqrt_scalar(a):
    i = lax.bitcast_convert_type(a, jnp.int32)
    i = jnp.int32(0x5F3759DF) - (i >> 1)
    y = lax.bitcast_convert_type(i, jnp.float32)
    half_a = 0.5 * a
    for _ in range(2):
        y = y * (1.5 - half_a * y * y)
    return y


_mesh = plsc.VectorSubcoreMesh(core_axis_name="c", subcore_axis_name="s")


@functools.partial(
    pl.kernel,
    mesh=_mesh,
    out_type=jax.ShapeDtypeStruct((N, D), jnp.float32),
    scratch_types=[
        pltpu.VMEM((NCHUNK, C), jnp.int32),
        pltpu.VMEM((C, D), jnp.float32),
        pltpu.VMEM((C, D), jnp.float32),
        pltpu.VMEM((C, D), jnp.float32),
        pltpu.VMEM((C, D), jnp.float32),
        pltpu.VMEM((D,), jnp.float32),
        pltpu.SemaphoreType.DMA,
        pltpu.SemaphoreType.DMA,
        pltpu.SemaphoreType.DMA,
        pltpu.SemaphoreType.DMA,
        pltpu.SemaphoreType.DMA,
        pltpu.SemaphoreType.DMA,
    ],
)
def _emb_kernel(idx_hbm, ww_hbm, wp_hbm, sc_hbm, out_hbm,
                idx_v, tok0_v, tok1_v, tok2_v, pos_v, scale_v,
                g0_sem, g1_sem, g2_sem, o0_sem, o1_sem, o2_sem):
    wid = lax.axis_index("s") * NC + lax.axis_index("c")
    t0 = wid * TW
    toks = (tok0_v, tok1_v, tok2_v)
    gsems = (g0_sem, g1_sem, g2_sem)
    osems = (o0_sem, o1_sem, o2_sem)

    def flat0(c):
        return (c & (B - 1)) * T + t0 + (c >> 2) * C

    pltpu.sync_copy(sc_hbm, scale_v)
    idx_copies = [
        pltpu.async_copy(idx_hbm.at[pl.ds(flat0(c), C)], idx_v.at[c], g2_sem)
        for c in range(NCHUNK)
    ]
    pltpu.sync_copy(wp_hbm.at[pl.ds(t0, C)], pos_v)
    for cp in idx_copies:
        cp.wait()

    def start_gather(c):
        return pltpu.async_copy(ww_hbm.at[idx_v.at[c]],
                                toks[c % 3], gsems[c % 3])

    def compute(c):
        tok_v = toks[c % 3]

        def row_body(r, cc):
            accs = [jnp.zeros((L,), jnp.float32) for _ in range(4)]
            for j in range(NJ):
                sl = pl.ds(j * L, L)
                x = tok_v[r, sl] + pos_v[r, sl]
                tok_v[r, sl] = x
                accs[j % 4] = accs[j % 4] + x * x
            acc = (accs[0] + accs[1]) + (accs[2] + accs[3])
            lanes = [acc[l] for l in range(L)]
            while len(lanes) > 1:
                lanes = [lanes[i] + lanes[i + 1]
                         for i in range(0, len(lanes), 2)]
            ms = lanes[0] * (1.0 / D) + EPS
            rstd = jnp.full((L,), _rsqrt_scalar(ms), jnp.float32)
            for j in range(NJ):
                sl = pl.ds(j * L, L)
                tok_v[r, sl] = tok_v[r, sl] * rstd * scale_v[sl]
            return cc

        lax.fori_loop(0, C, row_body, 0)

    gathers = {0: start_gather(0)}
    outs = {}
    for c in range(NCHUNK):
        if c + 1 < NCHUNK:
            if c - 2 in outs:
                outs[c - 2].wait()
            gathers[c + 1] = start_gather(c + 1)
        gathers[c].wait()
        if c == B:
            pltpu.sync_copy(wp_hbm.at[pl.ds(t0 + C, C)], pos_v)
        compute(c)
        outs[c] = pltpu.async_copy(toks[c % 3],
                                   out_hbm.at[pl.ds(flat0(c), C)],
                                   osems[c % 3])
    for c in range(max(0, NCHUNK - 3), NCHUNK):
        outs[c].wait()


def kernel(idx, W_word, W_pos, rms_scale):
    out = _emb_kernel(idx.reshape(N), W_word, W_pos, rms_scale)
    return out.reshape(B, T, D)
